# Initial kernel scaffold; baseline (speedup 1.0000x reference)
import jax
import jax.numpy as jnp
from jax.experimental import pallas as pl


def kernel(x, edge_index, W1, b1, W2, b2):
    raise NotImplementedError("write your pallas kernel here")


# speedup vs baseline: 22.0696x; 22.0696x over previous
import functools

import jax
import jax.numpy as jnp
from jax import lax
from jax.experimental import pallas as pl
from jax.experimental.pallas import tpu as pltpu
from jax.experimental.pallas import tpu_sc as plsc

N = 10000
E = 320000
D_IN = 128
D_H = 512

NCORE = 2
NSUB = 16
NSLICE = 4
FSL = D_H // NSLICE
PPC = NSLICE // NCORE

EPT = E // NSUB
CH = 128
SB = 8
NSB = 20
NCH = SB * NSB
EPT_PAD = NCH * CH

EPW = E // (NCORE * NSUB)
SSB = 10
SCH = SB * SSB
EPW_PAD = SCH * CH

TPT = 640
TROWS = NSUB * TPT

NB = 1000
NBLK = N // NB

_mesh = plsc.VectorSubcoreMesh(core_axis_name="c", subcore_axis_name="s")


@functools.partial(
    pl.kernel,
    out_type=jax.ShapeDtypeStruct((NCORE, NSUB, 1, TPT), jnp.float32),
    mesh=_mesh,
    scratch_types=[
        pltpu.VMEM((SB, CH), jnp.int32),
        pltpu.VMEM((CH,), jnp.float32),
        pltpu.VMEM((TPT,), jnp.float32),
        pltpu.VMEM_SHARED((TROWS,), jnp.float32),
    ],
)
def _count_kernel(dstp_hbm, cnt_hbm, idx_buf, ones_buf, zbuf, acc_sh):
    c = lax.axis_index("c")
    t = lax.axis_index("s")

    for g in range(CH // 16):
        ones_buf[pl.ds(g * 16, 16)] = jnp.full((16,), 1.0, jnp.float32)
    for g in range(TPT // 16):
        zbuf[pl.ds(g * 16, 16)] = jnp.zeros((16,), jnp.float32)

    pltpu.sync_copy(zbuf, acc_sh.at[pl.ds(t * TPT, TPT)])
    wbase = (c * NSUB + t) * SSB
    plsc.subcore_barrier()

    @pl.loop(0, SSB)
    def _(sblk):
        pltpu.sync_copy(dstp_hbm.at[wbase + sblk], idx_buf)
        for j in range(SB):
            pltpu.sync_copy(ones_buf, acc_sh.at[idx_buf.at[j]], add=True)

    plsc.subcore_barrier()
    pltpu.sync_copy(acc_sh.at[pl.ds(t * TPT, TPT)], cnt_hbm.at[c, t, 0])


def _mm_body(x_ref, w1_ref, cnt_ref, hs_ref, dinv_ref):
    deg = cnt_ref[0, 0, 0, :] + cnt_ref[1, 0, 0, :] + 1.0
    dinv = lax.rsqrt(deg)
    h = jnp.dot(x_ref[...], w1_ref[0], preferred_element_type=jnp.float32)
    hs_ref[...] = h * dinv[:, None]
    dinv_ref[0, 0, :] = dinv


def _run_mm(x, W1, cnt):
    return pl.pallas_call(
        _mm_body,
        grid=(NBLK, NSLICE),
        in_specs=[
            pl.BlockSpec((NB, D_IN), lambda i, j: (i, 0)),
            pl.BlockSpec((1, D_IN, FSL), lambda i, j: (j, 0, 0)),
            pl.BlockSpec((NCORE, 1, 1, NB), lambda i, j: (0, i, 0, 0)),
        ],
        out_specs=[
            pl.BlockSpec((NB, FSL), lambda i, j: (j * NBLK + i, 0)),
            pl.BlockSpec((1, 1, NB), lambda i, j: (i, 0, 0)),
        ],
        out_shape=[
            jax.ShapeDtypeStruct((NSLICE * N, FSL), jnp.float32),
            jax.ShapeDtypeStruct((NBLK, 1, NB), jnp.float32),
        ],
    )(x, W1, cnt)


@functools.partial(
    pl.kernel,
    out_type=[
        jax.ShapeDtypeStruct((NSLICE * TROWS, FSL), jnp.float32),
        jax.ShapeDtypeStruct((NCORE, NSUB, 1, TPT), jnp.float32),
    ],
    mesh=_mesh,
    scratch_types=[
        pltpu.VMEM((SB, CH), jnp.int32),
        pltpu.VMEM((SB, CH), jnp.int32),
        pltpu.VMEM((CH, FSL), jnp.float32),
        pltpu.VMEM((CH, FSL), jnp.float32),
        pltpu.VMEM((SB, CH), jnp.int32),
        pltpu.VMEM((SB, CH), jnp.int32),
        pltpu.VMEM((CH,), jnp.float32),
        pltpu.VMEM((TPT,), jnp.float32),
        pltpu.VMEM_SHARED((TROWS, FSL), jnp.float32),
        pltpu.VMEM_SHARED((TROWS,), jnp.float32),
        pltpu.SemaphoreType.DMA,
        pltpu.SemaphoreType.DMA,
    ],
)
def _agg_kernel(hs_hbm, srcp_hbm, dstp_hbm, dinv_hbm, ssrcp_hbm, sdstp_hbm,
                agg_hbm, sout_hbm,
                sbuf, dbuf, rows0, rows1, ssrc_buf, sdst_buf,
                supd_buf, zbuf, acc_sh, s_sh, g0, g1):
    c = lax.axis_index("c")
    t = lax.axis_index("s")

    for g in range(TPT // 16):
        zbuf[pl.ds(g * 16, 16)] = jnp.zeros((16,), jnp.float32)

    for it in range(PPC):
        base = (c * PPC + it) * N
        prow = (c * PPC + it) * TROWS

        @pl.loop(0, CH)
        def _(rr):
            for g in range(FSL // 16):
                rows0[rr, pl.ds(g * 16, 16)] = jnp.zeros((16,), jnp.float32)

        for z in range(TPT // CH):
            pltpu.sync_copy(
                rows0, acc_sh.at[pl.ds(t * TPT + z * CH, CH), :])
        if it == 0:
            pltpu.sync_copy(zbuf, s_sh.at[pl.ds(t * TPT, TPT)])
        plsc.subcore_barrier()

        if it == 0:
            swbase = (c * NSUB + t) * SSB

            @pl.loop(0, SSB)
            def _(sblk):
                pltpu.sync_copy(ssrcp_hbm.at[swbase + sblk], ssrc_buf)
                pltpu.sync_copy(sdstp_hbm.at[swbase + sblk], sdst_buf)
                for j in range(SB):
                    pltpu.sync_copy(dinv_hbm.at[sdst_buf.at[j]], supd_buf)
                    pltpu.sync_copy(supd_buf, s_sh.at[ssrc_buf.at[j]],
                                    add=True)

        tb = t * NSB

        @pl.loop(0, NSB)
        def _(sb):
            pltpu.sync_copy(srcp_hbm.at[tb + sb], sbuf)
            pltpu.sync_copy(dstp_hbm.at[tb + sb], dbuf)
            for j in range(SB):
                for g in range(CH // 16):
                    sl = pl.ds(g * 16, 16)
                    sbuf[j, sl] = sbuf[j, sl] + base

            pltpu.async_copy(hs_hbm.at[sbuf.at[0]], rows0, g0)
            pltpu.async_copy(hs_hbm.at[sbuf.at[1]], rows1, g1)
            for j in range(SB):
                buf = rows0 if j % 2 == 0 else rows1
                sem = g0 if j % 2 == 0 else g1
                pltpu.make_async_copy(hs_hbm.at[sbuf.at[j]], buf, sem).wait()
                pltpu.sync_copy(buf, acc_sh.at[dbuf.at[j]], add=True)
                if j + 2 < SB:
                    pltpu.async_copy(hs_hbm.at[sbuf.at[j + 2]], buf, sem)

        plsc.subcore_barrier()

        pltpu.sync_copy(
            acc_sh.at[pl.ds(t * TPT, TPT), :],
            agg_hbm.at[pl.ds(prow + t * TPT, TPT), :])
        if it == 0:
            pltpu.sync_copy(s_sh.at[pl.ds(t * TPT, TPT)],
                            sout_hbm.at[c, t, 0])

        plsc.subcore_barrier()


def _final_body(agg_ref, hs_ref, cnt_ref, s_ref, b1_ref, w2_ref, b2_ref,
                out_ref, racc):
    i = pl.program_id(0)

    @pl.when(i == 0)
    def _():
        racc[...] = jnp.zeros_like(racc)

    deg = cnt_ref[0, 0, 0, :] + cnt_ref[1, 0, 0, :] + 1.0
    dinv = lax.rsqrt(deg)
    s = s_ref[0, 0, 0, :] + s_ref[1, 0, 0, :]
    q = dinv * (s + dinv)

    for p in range(NSLICE):
        a1 = dinv[:, None] * (agg_ref[p] + hs_ref[p]) + b1_ref[p, :][None, :]
        h1 = jnp.maximum(a1, 0.0)
        racc[p, :] = racc[p, :] + jnp.sum(h1 * q[:, None], axis=0)

    @pl.when(i == NBLK - 1)
    def _():
        out = b2_ref[...]
        inv_n = 1.0 / N
        for p in range(NSLICE):
            out = out + jnp.dot(racc[p:p + 1, :] * inv_n, w2_ref[p],
                                preferred_element_type=jnp.float32)
        out_ref[...] = out


def _run_final(agg4, hs4, cnt, s_out, b1r, W2r, b2r):
    return pl.pallas_call(
        _final_body,
        grid=(NBLK,),
        in_specs=[
            pl.BlockSpec((NSLICE, NB, FSL), lambda i: (0, i, 0)),
            pl.BlockSpec((NSLICE, NB, FSL), lambda i: (0, i, 0)),
            pl.BlockSpec((NCORE, 1, 1, NB), lambda i: (0, i, 0, 0)),
            pl.BlockSpec((NCORE, 1, 1, NB), lambda i: (0, i, 0, 0)),
            pl.BlockSpec((NSLICE, FSL), lambda i: (0, 0)),
            pl.BlockSpec((NSLICE, FSL, D_H), lambda i: (0, 0, 0)),
            pl.BlockSpec((1, D_H), lambda i: (0, 0)),
        ],
        out_specs=pl.BlockSpec((1, D_H), lambda i: (0, 0)),
        out_shape=jax.ShapeDtypeStruct((1, D_H), jnp.float32),
        scratch_shapes=[pltpu.VMEM((NSLICE, FSL), jnp.float32)],
    )(agg4, hs4, cnt, s_out, b1r, W2r, b2r)


def kernel(x, edge_index, W1, b1, W2, b2):
    src = edge_index[0].astype(jnp.int32)
    dst = edge_index[1].astype(jnp.int32)

    pad_m = EPT_PAD - EPT
    trash = (N + (jnp.arange(pad_m, dtype=jnp.int32) % 8))[None, :]
    safe = (jnp.arange(pad_m, dtype=jnp.int32) % 8)[None, :]
    src_pre = jnp.concatenate(
        [src.reshape(NSUB, EPT), jnp.broadcast_to(safe, (NSUB, pad_m))],
        axis=1).reshape(NSUB * NSB, SB, CH)
    dst_pre = jnp.concatenate(
        [dst.reshape(NSUB, EPT), jnp.broadcast_to(trash, (NSUB, pad_m))],
        axis=1).reshape(NSUB * NSB, SB, CH)

    NW = NCORE * NSUB
    pad_s = EPW_PAD - EPW
    strash = (N + (jnp.arange(pad_s, dtype=jnp.int32) % 8))[None, :]
    ssafe = (jnp.arange(pad_s, dtype=jnp.int32) % 8)[None, :]
    dst_a_pre = jnp.concatenate(
        [dst.reshape(NW, EPW), jnp.broadcast_to(strash, (NW, pad_s))],
        axis=1).reshape(NW * SSB, SB, CH)
    ssrc_pre = jnp.concatenate(
        [src.reshape(NW, EPW), jnp.broadcast_to(strash, (NW, pad_s))],
        axis=1).reshape(NW * SSB, SB, CH)
    sdst_pre = jnp.concatenate(
        [dst.reshape(NW, EPW), jnp.broadcast_to(ssafe, (NW, pad_s))],
        axis=1).reshape(NW * SSB, SB, CH)

    cnt = _count_kernel(dst_a_pre)
    cnt4 = cnt.reshape(NCORE, TROWS)[:, :N].reshape(NCORE, NBLK, 1, NB)
    W1r = W1.reshape(D_IN, NSLICE, FSL).transpose(1, 0, 2)
    hs, dinv3 = _run_mm(x, W1r, cnt4)
    dinv = dinv3.reshape(N)
    agg, s_out = _agg_kernel(hs, src_pre, dst_pre, dinv, ssrc_pre, sdst_pre)

    out = _run_final(
        agg.reshape(NSLICE, TROWS, FSL), hs.reshape(NSLICE, N, FSL),
        cnt4,
        s_out.reshape(NCORE, TROWS)[:, :N].reshape(NCORE, NBLK, 1, NB),
        b1.reshape(NSLICE, FSL), W2.reshape(NSLICE, FSL, D_H),
        b2.reshape(1, D_H))
    return out
